# baseline (speedup 1.0000x reference)
import functools

import jax
import jax.numpy as jnp
from jax.experimental import pallas as pl
from jax.experimental.pallas import tpu as pltpu

B, C, H, W = 2, 96, 224, 224
F = 96
KH = KW = 3
OH = OW = 111
TH = 8
NT = 14
CHUNK = 576


def _conv_kernel(vals_ref, rows_ref, cols_ref, x1_ref, x2_ref, out_ref, w_ref,
                 sel_ref, *, nchunks):
    b = pl.program_id(0)
    t = pl.program_id(1)

    @pl.when(jnp.logical_and(b == 0, t == 0))
    def _build_weights():
        iota_f = jax.lax.broadcasted_iota(jnp.int32, (F, CHUNK), 0)
        iota_c = jax.lax.broadcasted_iota(jnp.int32, (CHUNK, C), 1)
        for tap in range(KH * KW):
            acc = jnp.zeros((F, C), jnp.float32)
            for ch in range(nchunks):
                rows_c = rows_ref[ch, :][None, :]
                vals_c = vals_ref[ch, :][None, :]
                cols_c = cols_ref[ch, :][:, None]
                sel = jnp.where(rows_c == iota_f, vals_c, 0.0)
                onehot = (cols_c == (iota_c + tap * C)).astype(jnp.float32)
                acc = acc + jnp.dot(sel, onehot,
                                    preferred_element_type=jnp.float32)
            w_ref[tap] = acc
        iw = jax.lax.broadcasted_iota(jnp.int32, (W, W), 0)
        iq = jax.lax.broadcasted_iota(jnp.int32, (W, W), 1)
        sel_ref[...] = jnp.where(
            iq < W // 2, (iw == 2 * iq).astype(jnp.bfloat16),
            (iw == 2 * (iq - W // 2) + 1).astype(jnp.bfloat16))

    def row(l):
        if l < 2 * TH:
            return x1_ref[0, :, l, :]
        return x2_ref[0, :, l - 2 * TH, :]

    sel = sel_ref[...]
    dei = [jnp.dot(row(l).astype(jnp.bfloat16), sel,
                   preferred_element_type=jnp.float32)
           for l in range(2 * TH + 1)]

    for r in range(TH):
        acc = jnp.zeros((F, OW), jnp.float32)
        for j in range(KH):
            d = dei[2 * r + j]
            ev = jax.lax.slice(d, (0, 0), (C, W // 2))
            od = jax.lax.slice(d, (0, W // 2), (C, W))
            acc = acc + jnp.dot(w_ref[j], jax.lax.slice(ev, (0, 0), (C, OW)),
                                preferred_element_type=jnp.float32)
            acc = acc + jnp.dot(w_ref[3 + j], jax.lax.slice(od, (0, 0), (C, OW)),
                                preferred_element_type=jnp.float32)
            acc = acc + jnp.dot(w_ref[6 + j], jax.lax.slice(ev, (0, 1), (C, OW + 1)),
                                preferred_element_type=jnp.float32)
        out_ref[0, :, r, :] = acc


def kernel(inputs, values, row_ids, col_idx):
    nnz = values.shape[0]
    nchunks = max(1, -(-nnz // CHUNK))
    pad = nchunks * CHUNK - nnz
    vals2 = jnp.pad(values, (0, pad)).reshape(nchunks, CHUNK)
    rows2 = jnp.pad(row_ids, (0, pad), constant_values=-1).reshape(nchunks, CHUNK)
    cols2 = jnp.pad(col_idx, (0, pad), constant_values=-1).reshape(nchunks, CHUNK)

    grid = (B, NT)
    outT = pl.pallas_call(
        functools.partial(_conv_kernel, nchunks=nchunks),
        grid=grid,
        in_specs=[
            pl.BlockSpec((nchunks, CHUNK), lambda b, t: (0, 0)),
            pl.BlockSpec((nchunks, CHUNK), lambda b, t: (0, 0)),
            pl.BlockSpec((nchunks, CHUNK), lambda b, t: (0, 0)),
            pl.BlockSpec((1, C, 2 * TH, W), lambda b, t: (b, 0, t, 0)),
            pl.BlockSpec((1, C, TH, W),
                         lambda b, t: (b, 0, jnp.minimum(2 * t + 2, 27), 0)),
        ],
        out_specs=pl.BlockSpec((1, F, TH, OW), lambda b, t: (b, 0, t, 0)),
        out_shape=jax.ShapeDtypeStruct((B, F, OH, OW), jnp.float32),
        scratch_shapes=[pltpu.VMEM((KH * KW, F, C), jnp.float32),
                        pltpu.VMEM((W, W), jnp.bfloat16)],
        compiler_params=pltpu.CompilerParams(
            dimension_semantics=("arbitrary", "arbitrary")),
    )(vals2, rows2, cols2, inputs, inputs)
    return jnp.swapaxes(outT, 2, 3)


# speedup vs baseline: 1.4251x; 1.4251x over previous
import functools

import jax
import jax.numpy as jnp
from jax.experimental import pallas as pl
from jax.experimental.pallas import tpu as pltpu

B, C, H, W = 2, 96, 224, 224
F = 96
KH = KW = 3
OH = OW = 111
TH = 16
NT = 7
CHUNK = 576


def _conv_kernel(vals_ref, rows_ref, cols_ref, x1_ref, x2_ref, out_ref, w_ref,
                 sel_ref, *, nchunks):
    b = pl.program_id(0)
    t = pl.program_id(1)

    @pl.when(jnp.logical_and(b == 0, t == 0))
    def _build_weights():
        iota_f = jax.lax.broadcasted_iota(jnp.int32, (F, CHUNK), 0)
        iota_c = jax.lax.broadcasted_iota(jnp.int32, (CHUNK, C), 1)
        for tap in range(KH * KW):
            acc = jnp.zeros((F, C), jnp.float32)
            for ch in range(nchunks):
                rows_c = rows_ref[ch, :][None, :]
                vals_c = vals_ref[ch, :][None, :]
                cols_c = cols_ref[ch, :][:, None]
                sel = jnp.where(rows_c == iota_f, vals_c, 0.0)
                onehot = (cols_c == (iota_c + tap * C)).astype(jnp.float32)
                acc = acc + jnp.dot(sel, onehot,
                                    preferred_element_type=jnp.float32)
            w_ref[tap] = acc
        iw = jax.lax.broadcasted_iota(jnp.int32, (W, W), 0)
        iq = jax.lax.broadcasted_iota(jnp.int32, (W, W), 1)
        sel_ref[...] = jnp.where(
            iq < W // 2, (iw == 2 * iq).astype(jnp.float32),
            (iw == 2 * (iq - W // 2) + 1).astype(jnp.float32))

    def row(l):
        if l < 2 * TH:
            return x1_ref[0, :, l, :]
        return x2_ref[0, :, l - 2 * TH, :]

    sel = sel_ref[...]
    dei = [jnp.dot(row(l), sel, preferred_element_type=jnp.float32)
           for l in range(2 * TH + 1)]

    for r in range(TH):
        acc = jnp.zeros((F, OW), jnp.float32)
        for j in range(KH):
            d = dei[2 * r + j]
            ev = jax.lax.slice(d, (0, 0), (C, W // 2))
            od = jax.lax.slice(d, (0, W // 2), (C, W))
            acc = acc + jnp.dot(w_ref[j], jax.lax.slice(ev, (0, 0), (C, OW)),
                                preferred_element_type=jnp.float32)
            acc = acc + jnp.dot(w_ref[3 + j], jax.lax.slice(od, (0, 0), (C, OW)),
                                preferred_element_type=jnp.float32)
            acc = acc + jnp.dot(w_ref[6 + j], jax.lax.slice(ev, (0, 1), (C, OW + 1)),
                                preferred_element_type=jnp.float32)
        out_ref[0, :, r, :] = acc


def kernel(inputs, values, row_ids, col_idx):
    nnz = values.shape[0]
    nchunks = max(1, -(-nnz // CHUNK))
    pad = nchunks * CHUNK - nnz
    vals2 = jnp.pad(values, (0, pad)).reshape(nchunks, CHUNK)
    rows2 = jnp.pad(row_ids, (0, pad), constant_values=-1).reshape(nchunks, CHUNK)
    cols2 = jnp.pad(col_idx, (0, pad), constant_values=-1).reshape(nchunks, CHUNK)

    grid = (B, NT)
    outT = pl.pallas_call(
        functools.partial(_conv_kernel, nchunks=nchunks),
        grid=grid,
        in_specs=[
            pl.BlockSpec((nchunks, CHUNK), lambda b, t: (0, 0)),
            pl.BlockSpec((nchunks, CHUNK), lambda b, t: (0, 0)),
            pl.BlockSpec((nchunks, CHUNK), lambda b, t: (0, 0)),
            pl.BlockSpec((1, C, 2 * TH, W), lambda b, t: (b, 0, t, 0)),
            pl.BlockSpec((1, C, 8, W),
                         lambda b, t: (b, 0, jnp.minimum(4 * t + 4, 27), 0)),
        ],
        out_specs=pl.BlockSpec((1, F, TH, OW), lambda b, t: (b, 0, t, 0)),
        out_shape=jax.ShapeDtypeStruct((B, F, OH, OW), jnp.float32),
        scratch_shapes=[pltpu.VMEM((KH * KW, F, C), jnp.float32),
                        pltpu.VMEM((W, W), jnp.float32)],
        compiler_params=pltpu.CompilerParams(
            dimension_semantics=("arbitrary", "arbitrary")),
    )(vals2, rows2, cols2, inputs, inputs)
    return jnp.swapaxes(outT, 2, 3)
